# baseline (speedup 1.0000x reference)
import functools

import jax
import jax.numpy as jnp
from jax import lax
from jax.experimental import pallas as pl
from jax.experimental.pallas import tpu as pltpu
from jax.experimental.pallas import tpu_sc as plsc

_L = 16
_ISTREAM = 128


@functools.lru_cache(maxsize=None)
def _make_gather(n_table_rows, dim, n_cols, n_samples):
    info = plsc.get_sparse_core_info()
    nw = info.num_cores * info.num_subcores
    spw = n_samples // nw
    n_streams = spw // _ISTREAM
    assert spw % _ISTREAM == 0 and n_cols % 2 == 0 and dim % _L == 0
    mesh = plsc.VectorSubcoreMesh(core_axis_name="c", subcore_axis_name="s")

    @functools.partial(
        pl.kernel,
        mesh=mesh,
        compiler_params=pltpu.CompilerParams(
            use_tc_tiling_on_sc=False, needs_layout_passes=False),
        out_type=jax.ShapeDtypeStruct(
            (n_cols, dim // 8, n_samples // 128, 8, 128), jnp.float32),
        scratch_types=[
            pltpu.VMEM((n_cols, spw), jnp.int32),
            pltpu.VMEM((spw, dim), jnp.float32),
            pltpu.VMEM((spw, dim), jnp.float32),
            pltpu.VMEM((dim, spw), jnp.float32),
            pltpu.VMEM((dim, spw), jnp.float32),
            *[pltpu.SemaphoreType.DMA for _ in range(4)],
        ],
    )
    def gather_kernel(table_hbm, idxT_hbm, outT_hbm, idx_v,
                      ga, gb, ta, tb, gsa, gsb, wsa, wsb):
        wid = lax.axis_index("s") * info.num_cores + lax.axis_index("c")
        s0 = wid * spw
        pltpu.sync_copy(idxT_hbm.at[:, pl.ds(s0, spw)], idx_v)

        def fire_gathers(gbuf, gsem, c):
            for k in range(n_streams):
                pltpu.async_copy(
                    table_hbm.at[idx_v.at[c, pl.ds(k * _ISTREAM, _ISTREAM)]],
                    gbuf.at[pl.ds(k * _ISTREAM, _ISTREAM)],
                    gsem,
                )

        def wait_gathers(gbuf, gsem):
            pltpu.make_async_copy(
                table_hbm.at[pl.ds(0, spw)], gbuf, gsem).wait()

        j0 = wid * (spw // 128)

        def fire_write(tbuf, wsem, c):
            for i in range(dim // 8):
                for jj in range(spw // 128):
                    pltpu.async_copy(
                        tbuf.at[pl.ds(i * 8, 8), pl.ds(jj * 128, 128)],
                        outT_hbm.at[c, i, j0 + jj], wsem)

        def wait_write(tbuf, wsem):
            for _ in range((dim // 8) * (spw // 128)):
                pltpu.make_async_copy(
                    tbuf.at[pl.ds(0, 8), pl.ds(0, 128)],
                    outT_hbm.at[0, 0, 0], wsem).wait()

        lane = lax.iota(jnp.int32, _L)
        rot = [jnp.bitwise_and(lane + k, _L - 1) for k in range(_L)]

        def transpose(gbuf, tbuf):
            def blk(v, carry):
                v0 = v * _L
                rl = lane + v0
                for h in range(dim // _L):
                    h16 = h * _L
                    for k in range(_L):
                        cidx = rot[k] + h16
                        vals = plsc.load_gather(gbuf, [rl, cidx])
                        plsc.store_scatter(tbuf, [cidx, rl], vals)
                return carry
            lax.fori_loop(0, spw // _L, blk, 0)

        fire_gathers(ga, gsa, 0)
        fire_gathers(gb, gsb, 1)

        def body(i, carry):
            c0 = i * 2
            c1 = c0 + 1
            wait_gathers(ga, gsa)

            @pl.when(i > 0)
            def _():
                wait_write(ta, wsa)
            transpose(ga, ta)
            fire_write(ta, wsa, c0)

            @pl.when(c0 + 2 < n_cols)
            def _():
                fire_gathers(ga, gsa, c0 + 2)

            wait_gathers(gb, gsb)

            @pl.when(i > 0)
            def _():
                wait_write(tb, wsb)
            transpose(gb, tb)
            fire_write(tb, wsb, c1)

            @pl.when(c1 + 2 < n_cols)
            def _():
                fire_gathers(gb, gsb, c1 + 2)
            return carry

        lax.fori_loop(0, n_cols // 2, body, 0)
        wait_write(ta, wsa)
        wait_write(tb, wsb)

    return gather_kernel


@functools.lru_cache(maxsize=None)
def _make_detranspose(dim, n_rows):
    info = plsc.get_sparse_core_info()
    nw = info.num_cores * info.num_subcores
    bw = 256
    n_blocks = n_rows // bw
    tail = n_rows - n_blocks * bw
    last_col = (n_blocks - 1) * bw
    obr = bw * dim // 128
    iters = (n_blocks + nw - 1) // nw
    pairs = (iters + 1) // 2
    mesh = plsc.VectorSubcoreMesh(core_axis_name="c", subcore_axis_name="s")

    @functools.partial(
        pl.kernel,
        mesh=mesh,
        compiler_params=pltpu.CompilerParams(needs_layout_passes=False),
        out_type=jax.ShapeDtypeStruct((n_rows * dim // 128, 128), jnp.float32),
        scratch_types=[
            pltpu.VMEM((dim, bw), jnp.float32),
            pltpu.VMEM((dim, bw), jnp.float32),
            pltpu.VMEM((obr, 128), jnp.float32),
            pltpu.VMEM((obr, 128), jnp.float32),
            pltpu.VMEM((dim, tail), jnp.float32),
            pltpu.VMEM((tail * dim // 128, 128), jnp.float32),
            *[pltpu.SemaphoreType.DMA for _ in range(4)],
        ],
    )
    def trans_kernel(tT_hbm, t4_hbm, ia, ib, oa, ob, tin, tout,
                     rsa, rsb, wsa, wsb):
        wid = lax.axis_index("s") * info.num_cores + lax.axis_index("c")

        def col_of(t):
            return pl.multiple_of(
                jnp.minimum((wid + nw * t) * bw, last_col), bw)

        def fire_read(ibuf, rsem, t):
            pltpu.async_copy(
                tT_hbm.at[:, pl.ds(col_of(t), bw)], ibuf, rsem)

        def wait_read(ibuf, rsem):
            pltpu.make_async_copy(
                tT_hbm.at[:, pl.ds(0, bw)], ibuf, rsem).wait()

        def fire_write(obuf, wsem, t):
            pltpu.async_copy(
                obuf,
                t4_hbm.at[pl.ds(pl.multiple_of(col_of(t) // 4, obr), obr)],
                wsem)

        def wait_write(obuf, wsem):
            pltpu.make_async_copy(
                obuf, t4_hbm.at[pl.ds(0, obr)], wsem).wait()

        lane = lax.iota(jnp.int32, _L)
        rot = [jnp.bitwise_and(lane + k, _L - 1) for k in range(_L)]
        dv = [lane + half * _L for half in range(dim // _L)]

        def transpose(ibuf, obuf, n_u):
            def ublk(ub, carry):
                u0 = ub * _L
                for k in range(_L):
                    uvec = rot[k] + u0
                    qvec = jax.lax.shift_right_logical(uvec, 2)
                    zbase = jax.lax.shift_left(
                        jnp.bitwise_and(uvec, 3), 5)
                    for half in range(dim // _L):
                        vals = plsc.load_gather(ibuf, [dv[half], uvec])
                        plsc.store_scatter(
                            obuf, [qvec, zbase + dv[half]], vals)
                return carry
            lax.fori_loop(0, n_u // _L, ublk, 0)

        fire_read(ia, rsa, 0)
        fire_read(ib, rsb, 1)

        def body(i, carry):
            t0 = 2 * i
            t1 = t0 + 1
            wait_read(ia, rsa)

            @pl.when(i > 0)
            def _():
                wait_write(oa, wsa)
            transpose(ia, oa, bw)
            fire_write(oa, wsa, t0)
            fire_read(ia, rsa, t0 + 2)

            wait_read(ib, rsb)

            @pl.when(i > 0)
            def _():
                wait_write(ob, wsb)
            transpose(ib, ob, bw)
            fire_write(ob, wsb, t1)
            fire_read(ib, rsb, t1 + 2)
            return carry

        lax.fori_loop(0, pairs, body, 0)
        wait_read(ia, rsa)
        wait_read(ib, rsb)
        wait_write(oa, wsa)
        wait_write(ob, wsb)

        if tail:
            @pl.when(wid == 0)
            def _():
                pltpu.sync_copy(
                    tT_hbm.at[:, pl.ds(n_blocks * bw, tail)], tin)
                transpose(tin, tout, tail)
                pltpu.sync_copy(
                    tout,
                    t4_hbm.at[pl.ds(n_blocks * obr, tail * dim // 128)])

    return trans_kernel


def kernel(label, table):
    n_samples, n_cols = label.shape
    n_rows, dim = table.shape
    t4 = _make_detranspose(dim, n_rows)(table.T)
    flat_table = t4.reshape(n_rows, dim)
    out = _make_gather(n_rows, dim, n_cols, n_samples)(flat_table, label.T)
    return out.transpose(2, 4, 0, 1, 3).reshape(n_samples, n_cols, dim)


# speedup vs baseline: 1.2088x; 1.2088x over previous
import functools

import jax
import jax.numpy as jnp
from jax import lax
from jax.experimental import pallas as pl
from jax.experimental.pallas import tpu as pltpu
from jax.experimental.pallas import tpu_sc as plsc

_L = 16
_ISTREAM = 128


@functools.lru_cache(maxsize=None)
def _make_gather(n_table_rows, dim, n_cols, n_samples):
    info = plsc.get_sparse_core_info()
    nw = info.num_cores * info.num_subcores
    spw = n_samples // nw
    n_streams = spw // _ISTREAM
    assert spw % _ISTREAM == 0 and n_cols % 2 == 0 and dim % _L == 0
    mesh = plsc.VectorSubcoreMesh(core_axis_name="c", subcore_axis_name="s")

    @functools.partial(
        pl.kernel,
        mesh=mesh,
        compiler_params=pltpu.CompilerParams(
            use_tc_tiling_on_sc=False, needs_layout_passes=False),
        out_type=jax.ShapeDtypeStruct(
            (n_cols, dim // 8, n_samples // 128, 8, 128), jnp.float32),
        scratch_types=[
            pltpu.VMEM((n_cols, spw), jnp.int32),
            pltpu.VMEM((spw, dim), jnp.float32),
            pltpu.VMEM((spw, dim), jnp.float32),
            pltpu.VMEM((dim, spw), jnp.float32),
            pltpu.VMEM((dim, spw), jnp.float32),
            *[pltpu.SemaphoreType.DMA for _ in range(4)],
        ],
    )
    def gather_kernel(table_hbm, idxT_hbm, outT_hbm, idx_v,
                      ga, gb, ta, tb, gsa, gsb, wsa, wsb):
        wid = lax.axis_index("s") * info.num_cores + lax.axis_index("c")
        s0 = wid * spw
        pltpu.sync_copy(idxT_hbm.at[:, pl.ds(s0, spw)], idx_v)

        def fire_gathers(gbuf, gsem, c):
            for k in range(n_streams):
                pltpu.async_copy(
                    table_hbm.at[idx_v.at[c, pl.ds(k * _ISTREAM, _ISTREAM)]],
                    gbuf.at[pl.ds(k * _ISTREAM, _ISTREAM)],
                    gsem,
                )

        def wait_gathers(gbuf, gsem):
            pltpu.make_async_copy(
                table_hbm.at[pl.ds(0, spw)], gbuf, gsem).wait()

        j0 = wid * (spw // 128)

        def fire_write(tbuf, wsem, c):
            for i in range(dim // 8):
                for jj in range(spw // 128):
                    pltpu.async_copy(
                        tbuf.at[pl.ds(i * 8, 8), pl.ds(jj * 128, 128)],
                        outT_hbm.at[c, i, j0 + jj], wsem)

        def wait_write(tbuf, wsem):
            for _ in range((dim // 8) * (spw // 128)):
                pltpu.make_async_copy(
                    tbuf.at[pl.ds(0, 8), pl.ds(0, 128)],
                    outT_hbm.at[0, 0, 0], wsem).wait()

        lane = lax.iota(jnp.int32, _L)
        rot = [jnp.bitwise_and(lane + k, _L - 1) for k in range(_L)]

        def transpose(gbuf, tbuf):
            @plsc.parallel_loop(0, spw // _L, unroll=2)
            def blk(v):
                v0 = v * _L
                rl = lane + v0
                for h in range(dim // _L):
                    h16 = h * _L
                    for k in range(_L):
                        cidx = rot[k] + h16
                        vals = plsc.load_gather(gbuf, [rl, cidx])
                        plsc.store_scatter(tbuf, [cidx, rl], vals)

        fire_gathers(ga, gsa, 0)
        fire_gathers(gb, gsb, 1)

        def body(i, carry):
            c0 = i * 2
            c1 = c0 + 1
            wait_gathers(ga, gsa)

            @pl.when(i > 0)
            def _():
                wait_write(ta, wsa)
            transpose(ga, ta)
            fire_write(ta, wsa, c0)

            @pl.when(c0 + 2 < n_cols)
            def _():
                fire_gathers(ga, gsa, c0 + 2)

            wait_gathers(gb, gsb)

            @pl.when(i > 0)
            def _():
                wait_write(tb, wsb)
            transpose(gb, tb)
            fire_write(tb, wsb, c1)

            @pl.when(c1 + 2 < n_cols)
            def _():
                fire_gathers(gb, gsb, c1 + 2)
            return carry

        lax.fori_loop(0, n_cols // 2, body, 0)
        wait_write(ta, wsa)
        wait_write(tb, wsb)

    return gather_kernel


@functools.lru_cache(maxsize=None)
def _make_detranspose(dim, n_rows):
    info = plsc.get_sparse_core_info()
    nw = info.num_cores * info.num_subcores
    bw = 256
    n_blocks = n_rows // bw
    tail = n_rows - n_blocks * bw
    last_col = (n_blocks - 1) * bw
    obr = bw * dim // 128
    iters = (n_blocks + nw - 1) // nw
    pairs = (iters + 1) // 2
    mesh = plsc.VectorSubcoreMesh(core_axis_name="c", subcore_axis_name="s")

    @functools.partial(
        pl.kernel,
        mesh=mesh,
        compiler_params=pltpu.CompilerParams(needs_layout_passes=False),
        out_type=jax.ShapeDtypeStruct((n_rows * dim // 128, 128), jnp.float32),
        scratch_types=[
            pltpu.VMEM((dim, bw), jnp.float32),
            pltpu.VMEM((dim, bw), jnp.float32),
            pltpu.VMEM((obr, 128), jnp.float32),
            pltpu.VMEM((obr, 128), jnp.float32),
            pltpu.VMEM((dim, tail), jnp.float32),
            pltpu.VMEM((tail * dim // 128, 128), jnp.float32),
            *[pltpu.SemaphoreType.DMA for _ in range(4)],
        ],
    )
    def trans_kernel(tT_hbm, t4_hbm, ia, ib, oa, ob, tin, tout,
                     rsa, rsb, wsa, wsb):
        wid = lax.axis_index("s") * info.num_cores + lax.axis_index("c")

        def col_of(t):
            return pl.multiple_of(
                jnp.minimum((wid + nw * t) * bw, last_col), bw)

        def fire_read(ibuf, rsem, t):
            pltpu.async_copy(
                tT_hbm.at[:, pl.ds(col_of(t), bw)], ibuf, rsem)

        def wait_read(ibuf, rsem):
            pltpu.make_async_copy(
                tT_hbm.at[:, pl.ds(0, bw)], ibuf, rsem).wait()

        def fire_write(obuf, wsem, t):
            pltpu.async_copy(
                obuf,
                t4_hbm.at[pl.ds(pl.multiple_of(col_of(t) // 4, obr), obr)],
                wsem)

        def wait_write(obuf, wsem):
            pltpu.make_async_copy(
                obuf, t4_hbm.at[pl.ds(0, obr)], wsem).wait()

        lane = lax.iota(jnp.int32, _L)
        rot = [jnp.bitwise_and(lane + k, _L - 1) for k in range(_L)]
        dv = [lane + half * _L for half in range(dim // _L)]

        def transpose(ibuf, obuf, n_u):
            @plsc.parallel_loop(0, n_u // _L, unroll=2)
            def ublk(ub):
                u0 = ub * _L
                for k in range(_L):
                    uvec = rot[k] + u0
                    qvec = jax.lax.shift_right_logical(uvec, 2)
                    zbase = jax.lax.shift_left(
                        jnp.bitwise_and(uvec, 3), 5)
                    for half in range(dim // _L):
                        vals = plsc.load_gather(ibuf, [dv[half], uvec])
                        plsc.store_scatter(
                            obuf, [qvec, zbase + dv[half]], vals)

        fire_read(ia, rsa, 0)
        fire_read(ib, rsb, 1)

        def body(i, carry):
            t0 = 2 * i
            t1 = t0 + 1
            wait_read(ia, rsa)

            @pl.when(i > 0)
            def _():
                wait_write(oa, wsa)
            transpose(ia, oa, bw)
            fire_write(oa, wsa, t0)
            fire_read(ia, rsa, t0 + 2)

            wait_read(ib, rsb)

            @pl.when(i > 0)
            def _():
                wait_write(ob, wsb)
            transpose(ib, ob, bw)
            fire_write(ob, wsb, t1)
            fire_read(ib, rsb, t1 + 2)
            return carry

        lax.fori_loop(0, pairs, body, 0)
        wait_read(ia, rsa)
        wait_read(ib, rsb)
        wait_write(oa, wsa)
        wait_write(ob, wsb)

        if tail:
            @pl.when(wid == 0)
            def _():
                pltpu.sync_copy(
                    tT_hbm.at[:, pl.ds(n_blocks * bw, tail)], tin)
                transpose(tin, tout, tail)
                pltpu.sync_copy(
                    tout,
                    t4_hbm.at[pl.ds(n_blocks * obr, tail * dim // 128)])

    return trans_kernel


def kernel(label, table):
    n_samples, n_cols = label.shape
    n_rows, dim = table.shape
    t4 = _make_detranspose(dim, n_rows)(table.T)
    flat_table = t4.reshape(n_rows, dim)
    out = _make_gather(n_rows, dim, n_cols, n_samples)(flat_table, label.T)
    return out.transpose(2, 4, 0, 1, 3).reshape(n_samples, n_cols, dim)


# speedup vs baseline: 1.7401x; 1.4396x over previous
import functools

import jax
import jax.numpy as jnp
from jax import lax
from jax.experimental import pallas as pl
from jax.experimental.pallas import tpu as pltpu
from jax.experimental.pallas import tpu_sc as plsc

_L = 16
_ISTREAM = 128


@functools.lru_cache(maxsize=None)
def _make_gather(n_table_rows, dim, n_cols, n_samples):
    info = plsc.get_sparse_core_info()
    nw = info.num_cores * info.num_subcores
    spw = n_samples // nw
    n_streams = spw // _ISTREAM
    assert spw % _ISTREAM == 0 and n_cols % 2 == 0 and dim % _L == 0
    mesh = plsc.VectorSubcoreMesh(core_axis_name="c", subcore_axis_name="s")

    @functools.partial(
        pl.kernel,
        mesh=mesh,
        compiler_params=pltpu.CompilerParams(
            use_tc_tiling_on_sc=False, needs_layout_passes=False),
        out_type=jax.ShapeDtypeStruct(
            (n_cols, dim // 8, n_samples // 128, 8, 128), jnp.float32),
        scratch_types=[
            pltpu.VMEM((n_cols, spw), jnp.int32),
            pltpu.VMEM((spw, dim), jnp.float32),
            pltpu.VMEM((spw, dim), jnp.float32),
            pltpu.VMEM((dim, spw), jnp.float32),
            pltpu.VMEM((dim, spw), jnp.float32),
            *[pltpu.SemaphoreType.DMA for _ in range(4)],
        ],
    )
    def gather_kernel(table_hbm, idxT_hbm, outT_hbm, idx_v,
                      ga, gb, ta, tb, gsa, gsb, wsa, wsb):
        wid = lax.axis_index("s") * info.num_cores + lax.axis_index("c")
        s0 = wid * spw
        pltpu.sync_copy(idxT_hbm.at[:, pl.ds(s0, spw)], idx_v)

        def fire_gathers(gbuf, gsem, c):
            for k in range(n_streams):
                pltpu.async_copy(
                    table_hbm.at[idx_v.at[c, pl.ds(k * _ISTREAM, _ISTREAM)]],
                    gbuf.at[pl.ds(k * _ISTREAM, _ISTREAM)],
                    gsem,
                )

        def wait_gathers(gbuf, gsem):
            pltpu.make_async_copy(
                table_hbm.at[pl.ds(0, spw)], gbuf, gsem).wait()

        j0 = wid * (spw // 128)

        def fire_write(tbuf, wsem, c):
            for i in range(dim // 8):
                for jj in range(spw // 128):
                    pltpu.async_copy(
                        tbuf.at[pl.ds(i * 8, 8), pl.ds(jj * 128, 128)],
                        outT_hbm.at[c, i, j0 + jj], wsem)

        def wait_write(tbuf, wsem):
            for _ in range((dim // 8) * (spw // 128)):
                pltpu.make_async_copy(
                    tbuf.at[pl.ds(0, 8), pl.ds(0, 128)],
                    outT_hbm.at[0, 0, 0], wsem).wait()

        lane = lax.iota(jnp.int32, _L)
        rot = [jnp.bitwise_and(lane + k, _L - 1) for k in range(_L)]

        def transpose(gbuf, tbuf):
            @plsc.parallel_loop(0, spw // _L, unroll=4)
            def blk(v):
                v0 = v * _L
                rl = lane + v0
                for h in range(dim // _L):
                    h16 = h * _L
                    for k in range(_L):
                        cidx = rot[k] + h16
                        vals = plsc.load_gather(gbuf, [rl, cidx])
                        plsc.store_scatter(tbuf, [cidx, rl], vals)

        fire_gathers(ga, gsa, 0)
        fire_gathers(gb, gsb, 1)

        def body(i, carry):
            c0 = i * 2
            c1 = c0 + 1
            wait_gathers(ga, gsa)

            @pl.when(i > 0)
            def _():
                wait_write(ta, wsa)
            transpose(ga, ta)
            fire_write(ta, wsa, c0)

            @pl.when(c0 + 2 < n_cols)
            def _():
                fire_gathers(ga, gsa, c0 + 2)

            wait_gathers(gb, gsb)

            @pl.when(i > 0)
            def _():
                wait_write(tb, wsb)
            transpose(gb, tb)
            fire_write(tb, wsb, c1)

            @pl.when(c1 + 2 < n_cols)
            def _():
                fire_gathers(gb, gsb, c1 + 2)
            return carry

        lax.fori_loop(0, n_cols // 2, body, 0)
        wait_write(ta, wsa)
        wait_write(tb, wsb)

    return gather_kernel


@functools.lru_cache(maxsize=None)
def _make_detranspose(dim, n_rows):
    info = plsc.get_sparse_core_info()
    nw = info.num_cores * info.num_subcores
    bw = 256
    n_blocks = n_rows // bw
    tail = n_rows - n_blocks * bw
    last_col = (n_blocks - 1) * bw
    obr = bw * dim // 128
    iters = (n_blocks + nw - 1) // nw
    pairs = (iters + 1) // 2
    mesh = plsc.VectorSubcoreMesh(core_axis_name="c", subcore_axis_name="s")

    @functools.partial(
        pl.kernel,
        mesh=mesh,
        compiler_params=pltpu.CompilerParams(needs_layout_passes=False),
        out_type=jax.ShapeDtypeStruct((n_rows * dim // 128, 128), jnp.float32),
        scratch_types=[
            pltpu.VMEM((dim, bw), jnp.float32),
            pltpu.VMEM((dim, bw), jnp.float32),
            pltpu.VMEM((obr, 128), jnp.float32),
            pltpu.VMEM((obr, 128), jnp.float32),
            pltpu.VMEM((dim, tail), jnp.float32),
            pltpu.VMEM((tail * dim // 128, 128), jnp.float32),
            *[pltpu.SemaphoreType.DMA for _ in range(4)],
        ],
    )
    def trans_kernel(tT_hbm, t4_hbm, ia, ib, oa, ob, tin, tout,
                     rsa, rsb, wsa, wsb):
        wid = lax.axis_index("s") * info.num_cores + lax.axis_index("c")

        def col_of(t):
            return pl.multiple_of(
                jnp.minimum((wid + nw * t) * bw, last_col), bw)

        def fire_read(ibuf, rsem, t):
            pltpu.async_copy(
                tT_hbm.at[:, pl.ds(col_of(t), bw)], ibuf, rsem)

        def wait_read(ibuf, rsem):
            pltpu.make_async_copy(
                tT_hbm.at[:, pl.ds(0, bw)], ibuf, rsem).wait()

        def fire_write(obuf, wsem, t):
            pltpu.async_copy(
                obuf,
                t4_hbm.at[pl.ds(pl.multiple_of(col_of(t) // 4, obr), obr)],
                wsem)

        def wait_write(obuf, wsem):
            pltpu.make_async_copy(
                obuf, t4_hbm.at[pl.ds(0, obr)], wsem).wait()

        lane = lax.iota(jnp.int32, _L)
        rot = [jnp.bitwise_and(lane + k, _L - 1) for k in range(_L)]
        dv = [lane + half * _L for half in range(dim // _L)]

        def transpose(ibuf, obuf, n_u):
            @plsc.parallel_loop(0, n_u // _L, unroll=4)
            def ublk(ub):
                u0 = ub * _L
                for k in range(_L):
                    uvec = rot[k] + u0
                    qvec = jax.lax.shift_right_logical(uvec, 2)
                    zbase = jax.lax.shift_left(
                        jnp.bitwise_and(uvec, 3), 5)
                    for half in range(dim // _L):
                        vals = plsc.load_gather(ibuf, [dv[half], uvec])
                        plsc.store_scatter(
                            obuf, [qvec, zbase + dv[half]], vals)

        fire_read(ia, rsa, 0)
        fire_read(ib, rsb, 1)

        def body(i, carry):
            t0 = 2 * i
            t1 = t0 + 1
            wait_read(ia, rsa)

            @pl.when(i > 0)
            def _():
                wait_write(oa, wsa)
            transpose(ia, oa, bw)
            fire_write(oa, wsa, t0)
            fire_read(ia, rsa, t0 + 2)

            wait_read(ib, rsb)

            @pl.when(i > 0)
            def _():
                wait_write(ob, wsb)
            transpose(ib, ob, bw)
            fire_write(ob, wsb, t1)
            fire_read(ib, rsb, t1 + 2)
            return carry

        lax.fori_loop(0, pairs, body, 0)
        wait_read(ia, rsa)
        wait_read(ib, rsb)
        wait_write(oa, wsa)
        wait_write(ob, wsb)

        if tail:
            @pl.when(wid == 0)
            def _():
                pltpu.sync_copy(
                    tT_hbm.at[:, pl.ds(n_blocks * bw, tail)], tin)
                transpose(tin, tout, tail)
                pltpu.sync_copy(
                    tout,
                    t4_hbm.at[pl.ds(n_blocks * obr, tail * dim // 128)])

    return trans_kernel


def kernel(label, table):
    n_samples, n_cols = label.shape
    n_rows, dim = table.shape
    t4 = _make_detranspose(dim, n_rows)(table.T)
    flat_table = t4.reshape(n_rows, dim)
    out = _make_gather(n_rows, dim, n_cols, n_samples)(flat_table, label.T)
    return out.transpose(2, 4, 0, 1, 3).reshape(n_samples, n_cols, dim)


# speedup vs baseline: 1.8506x; 1.0635x over previous
import functools

import jax
import jax.numpy as jnp
from jax import lax
from jax.experimental import pallas as pl
from jax.experimental.pallas import tpu as pltpu
from jax.experimental.pallas import tpu_sc as plsc

_L = 16
_ISTREAM = 128


@functools.lru_cache(maxsize=None)
def _make_gather(n_table_rows, dim, n_cols, n_samples):
    info = plsc.get_sparse_core_info()
    nw = info.num_cores * info.num_subcores
    spw = n_samples // nw
    n_streams = spw // _ISTREAM
    assert spw % _ISTREAM == 0 and n_cols % 2 == 0 and dim % _L == 0
    mesh = plsc.VectorSubcoreMesh(core_axis_name="c", subcore_axis_name="s")

    @functools.partial(
        pl.kernel,
        mesh=mesh,
        compiler_params=pltpu.CompilerParams(
            use_tc_tiling_on_sc=False, needs_layout_passes=False),
        out_type=jax.ShapeDtypeStruct(
            (n_cols, dim // 8, n_samples // 128, 8, 128), jnp.float32),
        scratch_types=[
            pltpu.VMEM((n_cols, spw), jnp.int32),
            pltpu.VMEM((spw, dim), jnp.float32),
            pltpu.VMEM((spw, dim), jnp.float32),
            pltpu.VMEM((dim, spw), jnp.float32),
            pltpu.VMEM((dim, spw), jnp.float32),
            *[pltpu.SemaphoreType.DMA for _ in range(4)],
        ],
    )
    def gather_kernel(table_hbm, idxT_hbm, outT_hbm, idx_v,
                      ga, gb, ta, tb, gsa, gsb, wsa, wsb):
        wid = lax.axis_index("s") * info.num_cores + lax.axis_index("c")
        s0 = wid * spw
        pltpu.sync_copy(idxT_hbm.at[:, pl.ds(s0, spw)], idx_v)

        def fire_gathers(gbuf, gsem, c):
            for k in range(n_streams):
                pltpu.async_copy(
                    table_hbm.at[idx_v.at[c, pl.ds(k * _ISTREAM, _ISTREAM)]],
                    gbuf.at[pl.ds(k * _ISTREAM, _ISTREAM)],
                    gsem,
                )

        def wait_gathers(gbuf, gsem):
            pltpu.make_async_copy(
                table_hbm.at[pl.ds(0, spw)], gbuf, gsem).wait()

        j0 = wid * (spw // 128)

        def fire_write(tbuf, wsem, c):
            for i in range(dim // 8):
                for jj in range(spw // 128):
                    pltpu.async_copy(
                        tbuf.at[pl.ds(i * 8, 8), pl.ds(jj * 128, 128)],
                        outT_hbm.at[c, i, j0 + jj], wsem)

        def wait_write(tbuf, wsem):
            for _ in range((dim // 8) * (spw // 128)):
                pltpu.make_async_copy(
                    tbuf.at[pl.ds(0, 8), pl.ds(0, 128)],
                    outT_hbm.at[0, 0, 0], wsem).wait()

        lane = lax.iota(jnp.int32, _L)
        rot = [jnp.bitwise_and(lane + k, _L - 1) for k in range(_L)]

        def transpose(gbuf, tbuf):
            @plsc.parallel_loop(0, spw // _L, unroll=8)
            def blk(v):
                v0 = v * _L
                rl = lane + v0
                for h in range(dim // _L):
                    h16 = h * _L
                    for k in range(_L):
                        cidx = rot[k] + h16
                        vals = plsc.load_gather(gbuf, [rl, cidx])
                        plsc.store_scatter(tbuf, [cidx, rl], vals)

        fire_gathers(ga, gsa, 0)
        fire_gathers(gb, gsb, 1)

        def body(i, carry):
            c0 = i * 2
            c1 = c0 + 1
            wait_gathers(ga, gsa)

            @pl.when(i > 0)
            def _():
                wait_write(ta, wsa)
            transpose(ga, ta)
            fire_write(ta, wsa, c0)

            @pl.when(c0 + 2 < n_cols)
            def _():
                fire_gathers(ga, gsa, c0 + 2)

            wait_gathers(gb, gsb)

            @pl.when(i > 0)
            def _():
                wait_write(tb, wsb)
            transpose(gb, tb)
            fire_write(tb, wsb, c1)

            @pl.when(c1 + 2 < n_cols)
            def _():
                fire_gathers(gb, gsb, c1 + 2)
            return carry

        lax.fori_loop(0, n_cols // 2, body, 0)
        wait_write(ta, wsa)
        wait_write(tb, wsb)

    return gather_kernel


@functools.lru_cache(maxsize=None)
def _make_detranspose(dim, n_rows):
    info = plsc.get_sparse_core_info()
    nw = info.num_cores * info.num_subcores
    bw = 256
    n_blocks = n_rows // bw
    tail = n_rows - n_blocks * bw
    last_col = (n_blocks - 1) * bw
    obr = bw * dim // 128
    iters = (n_blocks + nw - 1) // nw
    pairs = (iters + 1) // 2
    mesh = plsc.VectorSubcoreMesh(core_axis_name="c", subcore_axis_name="s")

    @functools.partial(
        pl.kernel,
        mesh=mesh,
        compiler_params=pltpu.CompilerParams(needs_layout_passes=False),
        out_type=jax.ShapeDtypeStruct((n_rows * dim // 128, 128), jnp.float32),
        scratch_types=[
            pltpu.VMEM((dim, bw), jnp.float32),
            pltpu.VMEM((dim, bw), jnp.float32),
            pltpu.VMEM((obr, 128), jnp.float32),
            pltpu.VMEM((obr, 128), jnp.float32),
            pltpu.VMEM((dim, tail), jnp.float32),
            pltpu.VMEM((tail * dim // 128, 128), jnp.float32),
            *[pltpu.SemaphoreType.DMA for _ in range(4)],
        ],
    )
    def trans_kernel(tT_hbm, t4_hbm, ia, ib, oa, ob, tin, tout,
                     rsa, rsb, wsa, wsb):
        wid = lax.axis_index("s") * info.num_cores + lax.axis_index("c")

        def col_of(t):
            return pl.multiple_of(
                jnp.minimum((wid + nw * t) * bw, last_col), bw)

        def fire_read(ibuf, rsem, t):
            pltpu.async_copy(
                tT_hbm.at[:, pl.ds(col_of(t), bw)], ibuf, rsem)

        def wait_read(ibuf, rsem):
            pltpu.make_async_copy(
                tT_hbm.at[:, pl.ds(0, bw)], ibuf, rsem).wait()

        def fire_write(obuf, wsem, t):
            pltpu.async_copy(
                obuf,
                t4_hbm.at[pl.ds(pl.multiple_of(col_of(t) // 4, obr), obr)],
                wsem)

        def wait_write(obuf, wsem):
            pltpu.make_async_copy(
                obuf, t4_hbm.at[pl.ds(0, obr)], wsem).wait()

        lane = lax.iota(jnp.int32, _L)
        rot = [jnp.bitwise_and(lane + k, _L - 1) for k in range(_L)]
        dv = [lane + half * _L for half in range(dim // _L)]

        def transpose(ibuf, obuf, n_u):
            @plsc.parallel_loop(0, n_u // _L, unroll=8)
            def ublk(ub):
                u0 = ub * _L
                for k in range(_L):
                    uvec = rot[k] + u0
                    qvec = jax.lax.shift_right_logical(uvec, 2)
                    zbase = jax.lax.shift_left(
                        jnp.bitwise_and(uvec, 3), 5)
                    for half in range(dim // _L):
                        vals = plsc.load_gather(ibuf, [dv[half], uvec])
                        plsc.store_scatter(
                            obuf, [qvec, zbase + dv[half]], vals)

        fire_read(ia, rsa, 0)
        fire_read(ib, rsb, 1)

        def body(i, carry):
            t0 = 2 * i
            t1 = t0 + 1
            wait_read(ia, rsa)

            @pl.when(i > 0)
            def _():
                wait_write(oa, wsa)
            transpose(ia, oa, bw)
            fire_write(oa, wsa, t0)
            fire_read(ia, rsa, t0 + 2)

            wait_read(ib, rsb)

            @pl.when(i > 0)
            def _():
                wait_write(ob, wsb)
            transpose(ib, ob, bw)
            fire_write(ob, wsb, t1)
            fire_read(ib, rsb, t1 + 2)
            return carry

        lax.fori_loop(0, pairs, body, 0)
        wait_read(ia, rsa)
        wait_read(ib, rsb)
        wait_write(oa, wsa)
        wait_write(ob, wsb)

        if tail:
            @pl.when(wid == 0)
            def _():
                pltpu.sync_copy(
                    tT_hbm.at[:, pl.ds(n_blocks * bw, tail)], tin)
                transpose(tin, tout, tail)
                pltpu.sync_copy(
                    tout,
                    t4_hbm.at[pl.ds(n_blocks * obr, tail * dim // 128)])

    return trans_kernel


def kernel(label, table):
    n_samples, n_cols = label.shape
    n_rows, dim = table.shape
    t4 = _make_detranspose(dim, n_rows)(table.T)
    flat_table = t4.reshape(n_rows, dim)
    out = _make_gather(n_rows, dim, n_cols, n_samples)(flat_table, label.T)
    return out.transpose(2, 4, 0, 1, 3).reshape(n_samples, n_cols, dim)
